# baseline (speedup 1.0000x reference)
import functools

import jax
import jax.numpy as jnp
from jax import lax
from jax.experimental import pallas as pl
from jax.experimental.pallas import tpu as pltpu
from jax.experimental.pallas import tpu_sc as plsc

NUM_EMBEDDINGS = 100001
EMBED_DIM = 32
BATCH = 16384

_info = plsc.get_sparse_core_info()
_NC, _NS, _NL = _info.num_cores, _info.num_subcores, _info.num_lanes
_NW = _NC * _NS
_B_PER_W = BATCH // _NW
_GROUP = 16
_NTILE_B = _B_PER_W // 128
_NTILE_D = EMBED_DIM // 8


def _make_gather():
    mesh = plsc.VectorSubcoreMesh(core_axis_name="c", subcore_axis_name="s")

    @functools.partial(
        pl.kernel,
        mesh=mesh,
        out_type=jax.ShapeDtypeStruct((EMBED_DIM, BATCH), jnp.float32),
        scratch_types=[
            pltpu.VMEM((_B_PER_W,), jnp.int32),
            pltpu.VMEM((_B_PER_W, EMBED_DIM), jnp.float32),
            [pltpu.VMEM((16, _B_PER_W), jnp.float32) for _ in range(2)],
            pltpu.SemaphoreType.DMA,
            pltpu.SemaphoreType.DMA,
        ],
        compiler_params=pltpu.CompilerParams(needs_layout_passes=False),
    )
    def gather_kernel(table_hbm, idx_hbm, out_hbm, idx_s, stage_v, tile_v, gsem, osem):
        wid = lax.axis_index("s") * _NC + lax.axis_index("c")
        base = wid * _B_PER_W
        pltpu.sync_copy(idx_hbm.at[pl.ds(base, _B_PER_W)], idx_s)

        def fire(g):
            qbase = g * _GROUP
            ivec = idx_s[pl.ds(qbase, _GROUP)]
            for j in range(_GROUP):
                pltpu.async_copy(
                    table_hbm.at[pl.ds(ivec[j], 1), :],
                    stage_v.at[pl.ds(qbase + j, 1), :],
                    gsem,
                )

        def drain(g):
            qbase = g * _GROUP
            pltpu.make_async_copy(
                table_hbm.at[pl.ds(0, _GROUP), :],
                stage_v.at[pl.ds(qbase, _GROUP), :],
                gsem,
            ).wait()

        lane = lax.iota(jnp.int32, _NL)

        def extract(g):
            for j in range(_GROUP):
                q = g * _GROUP + j
                bb = jnp.broadcast_to(q, (_NL,)).astype(jnp.int32)
                v_lo = stage_v[q, pl.ds(0, _NL)]
                v_hi = stage_v[q, pl.ds(_NL, _NL)]
                plsc.store_scatter(tile_v[0], [lane, bb], v_lo)
                plsc.store_scatter(tile_v[1], [lane, bb], v_hi)

        n_groups = _B_PER_W // _GROUP
        fire(0)
        fire(1)

        def step(g, _):
            fire(g)
            drain(g - 2)
            extract(g - 2)
            return ()

        lax.fori_loop(2, n_groups, step, (), unroll=False)
        for g in range(n_groups - 2, n_groups):
            drain(g)
            extract(g)
        for r in range(2):
            pltpu.async_copy(
                tile_v[r],
                out_hbm.at[pl.ds(16 * r, 16), pl.ds(base, _B_PER_W)],
                osem,
            ).wait()

    return gather_kernel


_gather = _make_gather()


def kernel(indices, table):
    return _gather(table, indices.astype(jnp.int32)).T


# speedup vs baseline: 1.0033x; 1.0033x over previous
import functools

import jax
import jax.numpy as jnp
from jax import lax
from jax.experimental import pallas as pl
from jax.experimental.pallas import tpu as pltpu
from jax.experimental.pallas import tpu_sc as plsc

NUM_EMBEDDINGS = 100001
EMBED_DIM = 32
BATCH = 16384

_info = plsc.get_sparse_core_info()
_NC, _NS, _NL = _info.num_cores, _info.num_subcores, _info.num_lanes
_NW = _NC * _NS
_B_PER_W = BATCH // _NW
_GROUP = 16
_NTILE_B = _B_PER_W // 128
_NTILE_D = EMBED_DIM // 8


def _make_gather():
    mesh = plsc.VectorSubcoreMesh(core_axis_name="c", subcore_axis_name="s")

    @functools.partial(
        pl.kernel,
        mesh=mesh,
        out_type=jax.ShapeDtypeStruct((EMBED_DIM, BATCH), jnp.float32),
        scratch_types=[
            pltpu.VMEM((_B_PER_W,), jnp.int32),
            pltpu.VMEM((_B_PER_W, EMBED_DIM), jnp.float32),
            [pltpu.VMEM((16, _B_PER_W), jnp.float32) for _ in range(2)],
            pltpu.SemaphoreType.DMA,
            pltpu.SemaphoreType.DMA,
        ],
        compiler_params=pltpu.CompilerParams(needs_layout_passes=False),
    )
    def gather_kernel(table_hbm, idx_hbm, out_hbm, idx_s, stage_v, tile_v, gsem, osem):
        wid = lax.axis_index("s") * _NC + lax.axis_index("c")
        base = wid * _B_PER_W
        pltpu.sync_copy(idx_hbm.at[pl.ds(base, _B_PER_W)], idx_s)

        def fire(g):
            qbase = g * _GROUP
            ivec = idx_s[pl.ds(qbase, _GROUP)]
            for j in range(_GROUP):
                pltpu.async_copy(
                    table_hbm.at[pl.ds(ivec[j], 1), :],
                    stage_v.at[pl.ds(qbase + j, 1), :],
                    gsem,
                )

        def drain(g):
            qbase = g * _GROUP
            pltpu.make_async_copy(
                table_hbm.at[pl.ds(0, _GROUP), :],
                stage_v.at[pl.ds(qbase, _GROUP), :],
                gsem,
            ).wait()

        lane = lax.iota(jnp.int32, _NL)

        def extract(g):
            for j in range(_GROUP):
                q = g * _GROUP + j
                bb = jnp.broadcast_to(q, (_NL,)).astype(jnp.int32)
                v_lo = stage_v[q, pl.ds(0, _NL)]
                v_hi = stage_v[q, pl.ds(_NL, _NL)]
                plsc.store_scatter(tile_v[0], [lane, bb], v_lo)
                plsc.store_scatter(tile_v[1], [lane, bb], v_hi)

        n_groups = _B_PER_W // _GROUP
        fire(0)
        fire(1)
        fire(2)

        def step(g, _):
            fire(g)
            drain(g - 3)
            extract(g - 3)
            return ()

        lax.fori_loop(3, n_groups, step, (), unroll=False)
        for g in range(n_groups - 3, n_groups):
            drain(g)
            extract(g)
        for r in range(2):
            pltpu.async_copy(
                tile_v[r],
                out_hbm.at[pl.ds(16 * r, 16), pl.ds(base, _B_PER_W)],
                osem,
            ).wait()

    return gather_kernel


_gather = _make_gather()


def kernel(indices, table):
    return _gather(table, indices.astype(jnp.int32)).T
